# baseline (speedup 1.0000x reference)
import functools

import jax
import jax.numpy as jnp
from jax import lax
from jax.experimental import pallas as pl
from jax.experimental.pallas import tpu as pltpu
from jax.experimental.pallas import tpu_sc as plsc

D_MODEL = 768
EPS = 1e-5
LANES = 16
NCORES = 2
NSUBCORES = 16
NWORKERS = NCORES * NSUBCORES
CHUNK = 64
NVEC = D_MODEL // LANES


def _allsum_vec(x):
    dn = lax.GatherDimensionNumbers(
        offset_dims=(), collapsed_slice_dims=(0,), start_index_map=(0,))
    iota = lax.iota(jnp.int32, LANES)
    for k in (8, 4, 2, 1):
        idx = (iota ^ k).reshape(LANES, 1)
        x = x + lax.gather(x, idx, dn, (1,),
                           mode=lax.GatherScatterMode.PROMISE_IN_BOUNDS)
    return x


def _rsqrt_vec(v):
    i = lax.bitcast_convert_type(v, jnp.int32)
    y = lax.bitcast_convert_type(jnp.int32(0x5F3759DF) - (i >> 1), jnp.float32)
    for _ in range(3):
        y = y * (1.5 - 0.5 * v * y * y)
    return y


def _make_sc_kernel(total_tokens):
    tok_per_w = total_tokens // NWORKERS
    nchunks = tok_per_w // CHUNK
    mesh = plsc.VectorSubcoreMesh(core_axis_name="c", subcore_axis_name="s")

    @functools.partial(
        pl.kernel,
        out_type=jax.ShapeDtypeStruct((total_tokens, D_MODEL), jnp.float32),
        mesh=mesh,
        scratch_types=[
            pltpu.VMEM((CHUNK,), jnp.int32),
            pltpu.VMEM((CHUNK, D_MODEL), jnp.float32),
            pltpu.VMEM((CHUNK, D_MODEL), jnp.float32),
            pltpu.VMEM((D_MODEL,), jnp.float32),
            pltpu.VMEM((D_MODEL,), jnp.float32),
            pltpu.SemaphoreType.DMA,
        ],
    )
    def k(ids_hbm, word_hbm, pos_hbm, gamma_hbm, beta_hbm, out_hbm,
          idx_v, rows_v, pos_v, g_v, b_v, sem):
        wid = lax.axis_index("s") * NCORES + lax.axis_index("c")
        tstart = wid * tok_per_w
        seq_len = pos_hbm.shape[0]

        pltpu.sync_copy(gamma_hbm, g_v)
        pltpu.sync_copy(beta_hbm, b_v)

        def chunk_body(c, _):
            tbase = tstart + c * CHUNK
            sbase = lax.rem(tbase, seq_len)
            pltpu.sync_copy(ids_hbm.at[pl.ds(tbase, CHUNK)], idx_v)
            pltpu.async_copy(word_hbm.at[idx_v], rows_v, sem).wait()
            pltpu.sync_copy(pos_hbm.at[pl.ds(sbase, CHUNK), :], pos_v)

            def tok_body(t, _):
                acc = jnp.zeros((LANES,), jnp.float32)
                acc2 = jnp.zeros((LANES,), jnp.float32)
                for j in range(NVEC):
                    x = rows_v[t, pl.ds(j * LANES, LANES)] + pos_v[t, pl.ds(j * LANES, LANES)]
                    acc = acc + x
                    acc2 = acc2 + x * x
                mean_v = _allsum_vec(acc) * (1.0 / D_MODEL)
                var_v = _allsum_vec(acc2) * (1.0 / D_MODEL) - mean_v * mean_v
                rs = _rsqrt_vec(var_v + EPS)
                for j in range(NVEC):
                    sl = pl.ds(j * LANES, LANES)
                    x = rows_v[t, sl] + pos_v[t, sl]
                    rows_v[t, sl] = (x - mean_v) * rs * g_v[sl] + b_v[sl]
                return 0

            lax.fori_loop(0, CHUNK, tok_body, 0)
            pltpu.sync_copy(rows_v, out_hbm.at[pl.ds(tbase, CHUNK), :])
            return 0

        lax.fori_loop(0, nchunks, chunk_body, 0)

    return k


def kernel(input_ids, word_table, pos_table, gamma, beta):
    batch, seq_len = input_ids.shape
    total = batch * seq_len
    ids_flat = input_ids.reshape(total).astype(jnp.int32)
    k = _make_sc_kernel(total)
    out = k(ids_flat, word_table, pos_table, gamma, beta)
    return out.reshape(batch, seq_len, word_table.shape[1])


# speedup vs baseline: 1.3337x; 1.3337x over previous
import functools

import jax
import jax.numpy as jnp
from jax import lax
from jax.experimental import pallas as pl
from jax.experimental.pallas import tpu as pltpu
from jax.experimental.pallas import tpu_sc as plsc

D_MODEL = 768
EPS = 1e-5
LANES = 16
NCORES = 2
NSUBCORES = 16
NWORKERS = NCORES * NSUBCORES
CHUNK = 32
GROUP = 4
NVEC = D_MODEL // LANES


def _allsum_vec(x):
    dn = lax.GatherDimensionNumbers(
        offset_dims=(), collapsed_slice_dims=(0,), start_index_map=(0,))
    iota = lax.iota(jnp.int32, LANES)
    for k in (8, 4, 2, 1):
        idx = (iota ^ k).reshape(LANES, 1)
        x = x + lax.gather(x, idx, dn, (1,),
                           mode=lax.GatherScatterMode.PROMISE_IN_BOUNDS)
    return x


def _rsqrt_vec(v):
    i = lax.bitcast_convert_type(v, jnp.int32)
    y = lax.bitcast_convert_type(jnp.int32(0x5F3759DF) - (i >> 1), jnp.float32)
    for _ in range(3):
        y = y * (1.5 - 0.5 * v * y * y)
    return y


def _make_sc_kernel(total_tokens):
    tok_per_w = total_tokens // NWORKERS
    nchunks = tok_per_w // CHUNK
    mesh = plsc.VectorSubcoreMesh(core_axis_name="c", subcore_axis_name="s")

    @functools.partial(
        pl.kernel,
        out_type=jax.ShapeDtypeStruct((total_tokens, D_MODEL), jnp.float32),
        mesh=mesh,
        scratch_types=[
            pltpu.VMEM((tok_per_w,), jnp.int32),
            pltpu.VMEM((3, CHUNK, D_MODEL), jnp.float32),
            pltpu.VMEM((2, CHUNK, D_MODEL), jnp.float32),
            pltpu.VMEM((D_MODEL,), jnp.float32),
            pltpu.VMEM((D_MODEL,), jnp.float32),
            pltpu.SemaphoreType.DMA((2,)),
            pltpu.SemaphoreType.DMA((2,)),
            pltpu.SemaphoreType.DMA((2,)),
        ],
    )
    def k(ids_hbm, word_hbm, pos_hbm, gamma_hbm, beta_hbm, out_hbm,
          idx_v, rows_v, pos_v, g_v, b_v, sem_g, sem_p, sem_o):
        wid = lax.axis_index("s") * NCORES + lax.axis_index("c")
        tstart = wid * tok_per_w
        seq_len = pos_hbm.shape[0]
        sstart = lax.rem(tstart, seq_len)

        pltpu.sync_copy(gamma_hbm, g_v)
        pltpu.sync_copy(beta_hbm, b_v)
        pltpu.sync_copy(ids_hbm.at[pl.ds(tstart, tok_per_w)], idx_v)

        def issue_in(c):
            p3 = lax.rem(c, 3)
            p2 = lax.rem(c, 2)
            pltpu.async_copy(
                word_hbm.at[idx_v.at[pl.ds(c * CHUNK, CHUNK)]],
                rows_v.at[p3], sem_g.at[p2])
            pltpu.async_copy(
                pos_hbm.at[pl.ds(sstart + c * CHUNK, CHUNK), :],
                pos_v.at[p2], sem_p.at[p2])

        def wait_in(c):
            p3 = lax.rem(c, 3)
            p2 = lax.rem(c, 2)
            pltpu.make_async_copy(
                word_hbm.at[idx_v.at[pl.ds(c * CHUNK, CHUNK)]],
                rows_v.at[p3], sem_g.at[p2]).wait()
            pltpu.make_async_copy(
                pos_hbm.at[pl.ds(sstart + c * CHUNK, CHUNK), :],
                pos_v.at[p2], sem_p.at[p2]).wait()

        def issue_out(c):
            p3 = lax.rem(c, 3)
            p2 = lax.rem(c, 2)
            pltpu.async_copy(
                rows_v.at[p3],
                out_hbm.at[pl.ds(tstart + c * CHUNK, CHUNK), :],
                sem_o.at[p2])

        def wait_out(c):
            p3 = lax.rem(c, 3)
            p2 = lax.rem(c, 2)
            pltpu.make_async_copy(
                rows_v.at[p3],
                out_hbm.at[pl.ds(tstart + c * CHUNK, CHUNK), :],
                sem_o.at[p2]).wait()

        issue_in(0)

        def chunk_body(c, _):
            p3 = lax.rem(c, 3)
            p2 = lax.rem(c, 2)

            @pl.when(c >= 2)
            def _():
                wait_out(c - 2)

            @pl.when(c + 1 < nchunks)
            def _():
                issue_in(c + 1)

            wait_in(c)

            def group_body(g, _):
                t0 = g * GROUP
                means = []
                rss = []
                for t in range(GROUP):
                    acc = jnp.zeros((LANES,), jnp.float32)
                    acc2 = jnp.zeros((LANES,), jnp.float32)
                    for j in range(NVEC):
                        sl = pl.ds(j * LANES, LANES)
                        x = rows_v[p3, t0 + t, sl] + pos_v[p2, t0 + t, sl]
                        rows_v[p3, t0 + t, sl] = x
                        acc = acc + x
                        acc2 = acc2 + x * x
                    mean_v = _allsum_vec(acc) * (1.0 / D_MODEL)
                    var_v = _allsum_vec(acc2) * (1.0 / D_MODEL) - mean_v * mean_v
                    means.append(mean_v)
                    rss.append(_rsqrt_vec(var_v + EPS))

                def jbody(j, _):
                    sl = pl.ds(j * LANES, LANES)
                    gj = g_v[sl]
                    bj = b_v[sl]
                    for t in range(GROUP):
                        x = rows_v[p3, t0 + t, sl]
                        rows_v[p3, t0 + t, sl] = (x - means[t]) * rss[t] * gj + bj
                    return 0

                lax.fori_loop(0, NVEC, jbody, 0)
                return 0

            lax.fori_loop(0, CHUNK // GROUP, group_body, 0)
            issue_out(c)
            return 0

        lax.fori_loop(0, nchunks, chunk_body, 0)
        wait_out(nchunks - 2)
        wait_out(nchunks - 1)

    return k


def kernel(input_ids, word_table, pos_table, gamma, beta):
    batch, seq_len = input_ids.shape
    total = batch * seq_len
    ids_flat = input_ids.reshape(total).astype(jnp.int32)
    k = _make_sc_kernel(total)
    out = k(ids_flat, word_table, pos_table, gamma, beta)
    return out.reshape(batch, seq_len, word_table.shape[1])


# speedup vs baseline: 1.9911x; 1.4928x over previous
import functools

import jax
import jax.numpy as jnp
from jax import lax
from jax.experimental import pallas as pl
from jax.experimental.pallas import tpu as pltpu
from jax.experimental.pallas import tpu_sc as plsc

D_MODEL = 768
EPS = 1e-5
NCORES = 2
NSUBCORES = 16
NWORKERS = NCORES * NSUBCORES
CHUNK = 32
TC_BLOCK = 512


def _make_sc_gather(total_tokens):
    tok_per_w = total_tokens // NWORKERS
    nchunks = tok_per_w // CHUNK
    mesh = plsc.VectorSubcoreMesh(core_axis_name="c", subcore_axis_name="s")

    @functools.partial(
        pl.kernel,
        out_type=jax.ShapeDtypeStruct((total_tokens, D_MODEL), jnp.float32),
        mesh=mesh,
        scratch_types=[
            pltpu.VMEM((tok_per_w,), jnp.int32),
            pltpu.VMEM((3, CHUNK, D_MODEL), jnp.float32),
            pltpu.SemaphoreType.DMA((2,)),
            pltpu.SemaphoreType.DMA((2,)),
        ],
    )
    def k(ids_hbm, word_hbm, out_hbm, idx_v, rows_v, sem_g, sem_o):
        wid = lax.axis_index("s") * NCORES + lax.axis_index("c")
        tstart = wid * tok_per_w
        pltpu.sync_copy(ids_hbm.at[pl.ds(tstart, tok_per_w)], idx_v)

        def gather_copy(c):
            return pltpu.make_async_copy(
                word_hbm.at[idx_v.at[pl.ds(c * CHUNK, CHUNK)]],
                rows_v.at[lax.rem(c, 3)], sem_g.at[lax.rem(c, 2)])

        def out_copy(c):
            return pltpu.make_async_copy(
                rows_v.at[lax.rem(c, 3)],
                out_hbm.at[pl.ds(tstart + c * CHUNK, CHUNK), :],
                sem_o.at[lax.rem(c, 2)])

        gather_copy(0).start()

        def chunk_body(c, _):
            @pl.when(c >= 2)
            def _():
                out_copy(c - 2).wait()

            @pl.when(c + 1 < nchunks)
            def _():
                gather_copy(c + 1).start()

            gather_copy(c).wait()
            out_copy(c).start()
            return 0

        lax.fori_loop(0, nchunks, chunk_body, 0)
        out_copy(nchunks - 2).wait()
        out_copy(nchunks - 1).wait()

    return k


def _tc_ln_body(g_ref, p_ref, gam_ref, bet_ref, o_ref):
    x = g_ref[...] + p_ref[...]
    mu = jnp.mean(x, axis=-1, keepdims=True)
    var = jnp.mean(x * x, axis=-1, keepdims=True) - mu * mu
    o_ref[...] = (x - mu) * lax.rsqrt(var + EPS) * gam_ref[...] + bet_ref[...]


def _make_tc_ln(seq_len):
    grid = seq_len // TC_BLOCK
    return pl.pallas_call(
        _tc_ln_body,
        grid=(grid,),
        in_specs=[
            pl.BlockSpec((TC_BLOCK, D_MODEL), lambda i: (i, 0)),
            pl.BlockSpec((TC_BLOCK, D_MODEL), lambda i: (i, 0)),
            pl.BlockSpec((1, D_MODEL), lambda i: (0, 0)),
            pl.BlockSpec((1, D_MODEL), lambda i: (0, 0)),
        ],
        out_specs=pl.BlockSpec((TC_BLOCK, D_MODEL), lambda i: (i, 0)),
        out_shape=jax.ShapeDtypeStruct((seq_len, D_MODEL), jnp.float32),
    )


def kernel(input_ids, word_table, pos_table, gamma, beta):
    batch, seq_len = input_ids.shape
    ids = input_ids.astype(jnp.int32)
    sc_gather = _make_sc_gather(seq_len)
    tc_ln = _make_tc_ln(seq_len)
    gam2 = gamma.reshape(1, D_MODEL)
    bet2 = beta.reshape(1, D_MODEL)
    outs = []
    for b in range(batch):
        rows = sc_gather(ids[b], word_table)
        outs.append(tc_ln(rows, pos_table, gam2, bet2))
    return jnp.stack(outs, axis=0)


# speedup vs baseline: 1.9971x; 1.0030x over previous
import functools

import jax
import jax.numpy as jnp
from jax import lax
from jax.experimental import pallas as pl
from jax.experimental.pallas import tpu as pltpu
from jax.experimental.pallas import tpu_sc as plsc

D_MODEL = 768
EPS = 1e-5
NCORES = 2
NSUBCORES = 16
NWORKERS = NCORES * NSUBCORES
CHUNK = 32
TC_BLOCK = 512


def _make_sc_gather(total_tokens):
    tok_per_w = total_tokens // NWORKERS
    nchunks = tok_per_w // CHUNK
    mesh = plsc.VectorSubcoreMesh(core_axis_name="c", subcore_axis_name="s")

    @functools.partial(
        pl.kernel,
        out_type=jax.ShapeDtypeStruct((total_tokens, D_MODEL), jnp.float32),
        mesh=mesh,
        scratch_types=[
            pltpu.VMEM((tok_per_w,), jnp.int32),
            pltpu.VMEM((3, CHUNK, D_MODEL), jnp.float32),
            pltpu.SemaphoreType.DMA((2,)),
            pltpu.SemaphoreType.DMA((2,)),
        ],
    )
    def k(ids_hbm, word_hbm, out_hbm, idx_v, rows_v, sem_g, sem_o):
        wid = lax.axis_index("s") * NCORES + lax.axis_index("c")
        tstart = wid * tok_per_w
        pltpu.sync_copy(ids_hbm.at[pl.ds(tstart, tok_per_w)], idx_v)

        def gather_copy(c):
            return pltpu.make_async_copy(
                word_hbm.at[idx_v.at[pl.ds(c * CHUNK, CHUNK)]],
                rows_v.at[lax.rem(c, 3)], sem_g.at[lax.rem(c, 2)])

        def out_copy(c):
            return pltpu.make_async_copy(
                rows_v.at[lax.rem(c, 3)],
                out_hbm.at[pl.ds(tstart + c * CHUNK, CHUNK), :],
                sem_o.at[lax.rem(c, 2)])

        gather_copy(0).start()

        def chunk_body(c, _):
            @pl.when(c >= 2)
            def _():
                out_copy(c - 2).wait()

            @pl.when(c + 1 < nchunks)
            def _():
                gather_copy(c + 1).start()

            gather_copy(c).wait()
            out_copy(c).start()
            return 0

        lax.fori_loop(0, nchunks, chunk_body, 0)
        out_copy(nchunks - 2).wait()
        out_copy(nchunks - 1).wait()

    return k


def _tc_ln_body(g_ref, p_ref, gam_ref, bet_ref, o_ref):
    x = g_ref[...] + p_ref[...]
    mu = jnp.mean(x, axis=-1, keepdims=True)
    var = jnp.mean(x * x, axis=-1, keepdims=True) - mu * mu
    o_ref[...] = (x - mu) * lax.rsqrt(var + EPS) * gam_ref[...] + bet_ref[...]


def _make_tc_ln(seq_len):
    grid = seq_len // TC_BLOCK
    return pl.pallas_call(
        _tc_ln_body,
        grid=(grid,),
        in_specs=[
            pl.BlockSpec((TC_BLOCK, D_MODEL), lambda i: (i, 0)),
            pl.BlockSpec((TC_BLOCK, D_MODEL), lambda i: (i, 0)),
            pl.BlockSpec((1, D_MODEL), lambda i: (0, 0)),
            pl.BlockSpec((1, D_MODEL), lambda i: (0, 0)),
        ],
        out_specs=pl.BlockSpec((TC_BLOCK, D_MODEL), lambda i: (i, 0)),
        out_shape=jax.ShapeDtypeStruct((seq_len, D_MODEL), jnp.float32),
    )


def kernel(input_ids, word_table, pos_table, gamma, beta):
    batch, seq_len = input_ids.shape
    ids = input_ids.astype(jnp.int32)
    sc_gather = _make_sc_gather(seq_len)
    tc_ln = _make_tc_ln(seq_len)
    gam2 = gamma.reshape(1, D_MODEL)
    bet2 = beta.reshape(1, D_MODEL)
    rows = [sc_gather(ids[b], word_table) for b in range(batch)]
    outs = [tc_ln(r, pos_table, gam2, bet2) for r in rows]
    return jnp.stack(outs, axis=0)


# speedup vs baseline: 2.8971x; 1.4506x over previous
import functools

import jax
import jax.numpy as jnp
from jax import lax
from jax.experimental import pallas as pl
from jax.experimental.pallas import tpu as pltpu
from jax.experimental.pallas import tpu_sc as plsc

D_MODEL = 768
EPS = 1e-5
NCORES = 2
NSUBCORES = 16
NWORKERS = NCORES * NSUBCORES
CHUNK = 32
TC_BLOCK = 512
NSLICES = 4


def _make_sc_gather(total_tokens):
    tok_per_w = total_tokens // NWORKERS
    nchunks = tok_per_w // CHUNK
    mesh = plsc.VectorSubcoreMesh(core_axis_name="c", subcore_axis_name="s")

    @functools.partial(
        pl.kernel,
        out_type=jax.ShapeDtypeStruct((total_tokens, D_MODEL), jnp.float32),
        mesh=mesh,
        scratch_types=[
            pltpu.VMEM((tok_per_w,), jnp.int32),
            pltpu.VMEM((3, CHUNK, D_MODEL), jnp.float32),
            pltpu.SemaphoreType.DMA((2,)),
            pltpu.SemaphoreType.DMA((2,)),
        ],
    )
    def k(ids_hbm, word_hbm, out_hbm, idx_v, rows_v, sem_g, sem_o):
        wid = lax.axis_index("s") * NCORES + lax.axis_index("c")
        tstart = wid * tok_per_w
        pltpu.sync_copy(ids_hbm.at[pl.ds(tstart, tok_per_w)], idx_v)

        def gather_copy(c):
            return pltpu.make_async_copy(
                word_hbm.at[idx_v.at[pl.ds(c * CHUNK, CHUNK)]],
                rows_v.at[lax.rem(c, 3)], sem_g.at[lax.rem(c, 2)])

        def out_copy(c):
            return pltpu.make_async_copy(
                rows_v.at[lax.rem(c, 3)],
                out_hbm.at[pl.ds(tstart + c * CHUNK, CHUNK), :],
                sem_o.at[lax.rem(c, 2)])

        gather_copy(0).start()

        def chunk_body(c, _):
            @pl.when(c >= 2)
            def _():
                out_copy(c - 2).wait()

            @pl.when(c + 1 < nchunks)
            def _():
                gather_copy(c + 1).start()

            gather_copy(c).wait()
            out_copy(c).start()
            return 0

        lax.fori_loop(0, nchunks, chunk_body, 0)
        out_copy(nchunks - 2).wait()
        out_copy(nchunks - 1).wait()

    return k


def _tc_ln_body_first(rows_ref, p_ref, gam_ref, bet_ref, o_ref):
    x = rows_ref[0] + p_ref[...]
    mu = jnp.mean(x, axis=-1, keepdims=True)
    var = jnp.mean(x * x, axis=-1, keepdims=True) - mu * mu
    o_ref[0] = (x - mu) * lax.rsqrt(var + EPS) * gam_ref[...] + bet_ref[...]


def _tc_ln_body(rows_ref, p_ref, gam_ref, bet_ref, big_ref, o_ref):
    del big_ref
    _tc_ln_body_first(rows_ref, p_ref, gam_ref, bet_ref, o_ref)


def _make_tc_ln(batch, seq_len, slice_idx, first):
    s_per_slice = seq_len // NSLICES
    nsb = s_per_slice // TC_BLOCK
    sb0 = slice_idx * nsb

    in_specs = [
        pl.BlockSpec((1, TC_BLOCK, D_MODEL), lambda s, b: (b, s, 0)),
        pl.BlockSpec((TC_BLOCK, D_MODEL), lambda s, b: (sb0 + s, 0)),
        pl.BlockSpec((1, D_MODEL), lambda s, b: (0, 0)),
        pl.BlockSpec((1, D_MODEL), lambda s, b: (0, 0)),
    ]
    if first:
        body = _tc_ln_body_first
        aliases = {}
    else:
        body = _tc_ln_body
        in_specs.append(pl.BlockSpec(memory_space=pl.ANY))
        aliases = {4: 0}
    return pl.pallas_call(
        body,
        grid=(nsb, batch),
        in_specs=in_specs,
        out_specs=pl.BlockSpec((1, TC_BLOCK, D_MODEL),
                               lambda s, b: (b, sb0 + s, 0)),
        out_shape=jax.ShapeDtypeStruct((batch, seq_len, D_MODEL), jnp.float32),
        input_output_aliases=aliases,
    )


def kernel(input_ids, word_table, pos_table, gamma, beta):
    batch, seq_len = input_ids.shape
    ids = input_ids.astype(jnp.int32)
    s_per_slice = seq_len // NSLICES
    sc_gather = _make_sc_gather(batch * s_per_slice)
    gam2 = gamma.reshape(1, D_MODEL)
    bet2 = beta.reshape(1, D_MODEL)

    rows = []
    for k in range(NSLICES):
        ids_k = lax.slice_in_dim(ids, k * s_per_slice, (k + 1) * s_per_slice,
                                 axis=1).reshape(batch * s_per_slice)
        rows.append(sc_gather(ids_k, word_table)
                    .reshape(batch, s_per_slice, D_MODEL))

    big = _make_tc_ln(batch, seq_len, 0, True)(rows[0], pos_table, gam2, bet2)
    for k in range(1, NSLICES):
        big = _make_tc_ln(batch, seq_len, k, False)(
            rows[k], pos_table, gam2, bet2, big)
    return big
